# baseline (speedup 1.0000x reference)
import functools

import jax
import jax.numpy as jnp
from jax import lax
from jax.experimental import pallas as pl
from jax.experimental.pallas import tpu as pltpu
from jax.experimental.pallas import tpu_sc as plsc

_VB = 768


def _sc_gather(idx, table):
    (B,) = idx.shape
    V, D = table.shape
    info = plsc.get_sparse_core_info()
    NC, NS = info.num_cores, info.num_subcores
    NW = NC * NS
    b_per_w = B // NW
    mesh = plsc.VectorSubcoreMesh(core_axis_name="c", subcore_axis_name="s")

    @functools.partial(
        pl.kernel,
        mesh=mesh,
        out_type=jax.ShapeDtypeStruct((B, D), jnp.float32),
        scratch_types=[
            pltpu.VMEM((b_per_w,), jnp.int32),
            pltpu.VMEM((b_per_w, D), jnp.float32),
            pltpu.SemaphoreType.DMA,
        ],
    )
    def gather_kernel(idx_hbm, table_hbm, out_hbm, idx_v, rows_v, sem):
        wid = lax.axis_index("s") * NC + lax.axis_index("c")
        base = wid * b_per_w
        pltpu.sync_copy(idx_hbm.at[pl.ds(base, b_per_w)], idx_v)
        pltpu.async_copy(table_hbm.at[idx_v], rows_v, sem).wait()
        pltpu.sync_copy(rows_v, out_hbm.at[pl.ds(base, b_per_w)])

    return gather_kernel(idx, table)


def _mm_body(x_ref, pos_ref, w_ref, out_ref, xs_ref):
    @pl.when(pl.program_id(0) == 0)
    def _():
        xs_ref[...] = (x_ref[...] + pos_ref[...]).astype(jnp.bfloat16)

    out_ref[:, 0, :] = lax.dot_general(
        w_ref[...].astype(jnp.bfloat16),
        xs_ref[...],
        (((1,), (1,)), ((), ())),
        preferred_element_type=jnp.float32,
    )


def _mm_t(x, pos, W_out):
    S, E = x.shape
    V = W_out.shape[0]
    n_tiles = pl.cdiv(V, _VB)
    return pl.pallas_call(
        _mm_body,
        grid=(n_tiles,),
        in_specs=[
            pl.BlockSpec((S, E), lambda i: (0, 0)),
            pl.BlockSpec((S, E), lambda i: (0, 0)),
            pl.BlockSpec((_VB, E), lambda i: (i, 0)),
        ],
        out_specs=pl.BlockSpec((_VB, 1, S), lambda i: (i, 0, 0)),
        out_shape=jax.ShapeDtypeStruct((V, 1, S), jnp.float32),
        scratch_shapes=[pltpu.VMEM((S, E), jnp.bfloat16)],
    )(x, pos, W_out)


def kernel(in_idx, tok_emb, pos_emb, W_out):
    B, S = in_idx.shape
    V, E = tok_emb.shape
    tok = _sc_gather(in_idx.reshape(-1), tok_emb)
    logits_t = _mm_t(tok, pos_emb[:S], W_out)
    return jnp.transpose(logits_t, (1, 2, 0))


# speedup vs baseline: 1.1395x; 1.1395x over previous
import functools

import jax
import jax.numpy as jnp
from jax import lax
from jax.experimental import pallas as pl
from jax.experimental.pallas import tpu as pltpu
from jax.experimental.pallas import tpu_sc as plsc

_VB = 1024


def _sc_gather(idx, table):
    (B,) = idx.shape
    V, D = table.shape
    info = plsc.get_sparse_core_info()
    NC, NS = info.num_cores, info.num_subcores
    NW = NC * NS
    b_per_w = B // NW
    mesh = plsc.VectorSubcoreMesh(core_axis_name="c", subcore_axis_name="s")

    @functools.partial(
        pl.kernel,
        mesh=mesh,
        out_type=jax.ShapeDtypeStruct((B, D), jnp.float32),
        scratch_types=[
            pltpu.VMEM((b_per_w,), jnp.int32),
            pltpu.VMEM((b_per_w, D), jnp.float32),
            pltpu.SemaphoreType.DMA,
        ],
    )
    def gather_kernel(idx_hbm, table_hbm, out_hbm, idx_v, rows_v, sem):
        wid = lax.axis_index("s") * NC + lax.axis_index("c")
        base = wid * b_per_w
        pltpu.sync_copy(idx_hbm.at[pl.ds(base, b_per_w)], idx_v)
        pltpu.async_copy(table_hbm.at[idx_v], rows_v, sem).wait()
        pltpu.sync_copy(rows_v, out_hbm.at[pl.ds(base, b_per_w)])

    return gather_kernel(idx, table)


def _mm_body(x_ref, pos_ref, w_ref, out_hbm, xs_ref, acc0, acc1, sem0, sem1):
    i = pl.program_id(0)
    n = pl.num_programs(0)
    VB, S = acc0.shape

    @pl.when(i == 0)
    def _():
        xs_ref[...] = (x_ref[...] + pos_ref[...]).astype(jnp.bfloat16)

    def step(acc, sem):
        @pl.when(i >= 2)
        def _():
            pltpu.make_async_copy(
                acc, out_hbm.at[pl.ds((i - 2) * VB, VB), 0, :], sem
            ).wait()

        acc[...] = lax.dot_general(
            w_ref[...].astype(jnp.bfloat16),
            xs_ref[...],
            (((1,), (1,)), ((), ())),
            preferred_element_type=jnp.float32,
        )
        pltpu.make_async_copy(
            acc, out_hbm.at[pl.ds(i * VB, VB), 0, :], sem
        ).start()

    @pl.when(i % 2 == 0)
    def _():
        step(acc0, sem0)

    @pl.when(i % 2 == 1)
    def _():
        step(acc1, sem1)

    @pl.when(i == n - 1)
    def _():
        prev_acc, prev_sem = (acc0, sem0) if (n - 2) % 2 == 0 else (acc1, sem1)
        last_acc, last_sem = (acc0, sem0) if (n - 1) % 2 == 0 else (acc1, sem1)
        pltpu.make_async_copy(
            prev_acc, out_hbm.at[pl.ds((n - 2) * VB, VB), 0, :], prev_sem
        ).wait()
        pltpu.make_async_copy(
            last_acc, out_hbm.at[pl.ds((n - 1) * VB, VB), 0, :], last_sem
        ).wait()


def _mm_t(x, pos, W_out):
    S, E = x.shape
    V = W_out.shape[0]
    n_full = V // _VB
    return pl.pallas_call(
        _mm_body,
        grid=(n_full,),
        in_specs=[
            pl.BlockSpec((S, E), lambda i: (0, 0)),
            pl.BlockSpec((S, E), lambda i: (0, 0)),
            pl.BlockSpec((_VB, E), lambda i: (i, 0)),
        ],
        out_specs=pl.BlockSpec(memory_space=pltpu.MemorySpace.HBM),
        out_shape=jax.ShapeDtypeStruct((V, 1, S), jnp.float32),
        scratch_shapes=[
            pltpu.VMEM((S, E), jnp.bfloat16),
            pltpu.VMEM((_VB, S), jnp.float32),
            pltpu.VMEM((_VB, S), jnp.float32),
            pltpu.SemaphoreType.DMA,
            pltpu.SemaphoreType.DMA,
        ],
    )(x, pos, W_out)


def _tail_body(main_ref, x_ref, pos_ref, w_ref, out_ref):
    xb = (x_ref[...] + pos_ref[...]).astype(jnp.bfloat16)
    out_ref[:, 0, :] = lax.dot_general(
        w_ref[...].astype(jnp.bfloat16),
        xb,
        (((1,), (1,)), ((), ())),
        preferred_element_type=jnp.float32,
    )


def _mm_tail(main_out, x, pos, W_out):
    S, E = x.shape
    V = W_out.shape[0]
    n_full = V // _VB
    return pl.pallas_call(
        _tail_body,
        grid=(1,),
        in_specs=[
            pl.BlockSpec(memory_space=pltpu.MemorySpace.HBM),
            pl.BlockSpec((S, E), lambda i: (0, 0)),
            pl.BlockSpec((S, E), lambda i: (0, 0)),
            pl.BlockSpec((_VB, E), lambda i: (n_full, 0)),
        ],
        out_specs=pl.BlockSpec((_VB, 1, S), lambda i: (n_full, 0, 0)),
        out_shape=jax.ShapeDtypeStruct((V, 1, S), jnp.float32),
        input_output_aliases={0: 0},
    )(main_out, x, pos, W_out)


def kernel(in_idx, tok_emb, pos_emb, W_out):
    B, S = in_idx.shape
    V, E = tok_emb.shape
    tok = _sc_gather(in_idx.reshape(-1), tok_emb)
    main = _mm_t(tok, pos_emb[:S], W_out)
    logits_t = _mm_tail(main, tok, pos_emb[:S], W_out)
    return jnp.transpose(logits_t, (1, 2, 0))
